# baseline (speedup 1.0000x reference)
import functools
import math

import jax
import jax.numpy as jnp
from jax.experimental import pallas as pl
from jax.experimental.pallas import tpu as pltpu

_LANES = 128
_SUBLANES = 8
_STRIP = 128
_CORES = 2
_ROW_QUANT = 2048
_TILE_CANDIDATES = (9216, 8192, 6144, 4608, 4096, 3072, 2048)


def _scalar_outputs():
    return (
        [
            pl.BlockSpec((None, 1, 1), lambda c, i: (c, 0, 0),
                         memory_space=pltpu.SMEM),
            pl.BlockSpec((None, 1, 1), lambda c, i: (c, 0, 0),
                         memory_space=pltpu.SMEM),
        ],
        [
            jax.ShapeDtypeStruct((_CORES, 1, 1), jnp.float32),
            jax.ShapeDtypeStruct((_CORES, 1, 1), jnp.float32),
        ],
    )


def _finalize(num_p, den_p, w_l1):
    num = num_p[0, 0, 0] + num_p[1, 0, 0]
    den = den_p[0, 0, 0] + den_p[1, 0, 0]
    return {'losses': {'depth_loss': (w_l1 * num) / (den + 1e-6)}}


def _accum_strips(p_ref, l_ref, acc_ref, index, n_strips, width):
    folds = _STRIP // _SUBLANES
    lgroups = width // _LANES

    def fold(x):
        s = x.reshape(folds, _SUBLANES, width).sum(axis=0)
        if lgroups == 1:
            return s
        return sum(s[:, g * _LANES:(g + 1) * _LANES] for g in range(lgroups))

    def strip(k, carry):
        acc_n, acc_d = carry
        p = index(p_ref, k).astype(jnp.float32)
        l = index(l_ref, k).astype(jnp.float32)
        valid = l > 1e-6
        l1 = jnp.where(valid, jnp.abs(p - l), 0.0)
        one = jnp.where(valid, 1.0, 0.0)
        return acc_n + fold(l1), acc_d + fold(one)

    zero = jnp.zeros((_SUBLANES, _LANES), jnp.float32)
    acc_n, acc_d = jax.lax.fori_loop(0, n_strips, strip, (zero, zero))
    acc_ref[0] += acc_n
    acc_ref[1] += acc_d


def _native_kernel(p_ref, l_ref, loss_ref, acc_ref, *,
                   strips_per_img, h, w, imgs, w_l1):
    step = pl.program_id(0)

    @pl.when(step == 0)
    def _init():
        acc_ref[...] = jnp.zeros_like(acc_ref)

    def index(ref, k):
        img = k // strips_per_img
        r0 = pl.multiple_of((k % strips_per_img) * _STRIP, _STRIP)
        return ref[img, pl.ds(r0, _STRIP), :]

    _accum_strips(p_ref, l_ref, acc_ref, index,
                  imgs * strips_per_img, w)

    @pl.when(step == pl.num_programs(0) - 1)
    def _finish():
        num = jnp.sum(acc_ref[0]) * w_l1
        den = jnp.sum(acc_ref[1])
        loss_ref[0, 0] = num / (den + 1e-6)


def _flat_kernel(p_ref, l_ref, num_ref, den_ref, acc_ref, *,
                 n_strips, w_l1):
    step = pl.program_id(1)

    @pl.when(step == 0)
    def _init():
        acc_ref[...] = jnp.zeros_like(acc_ref)

    def index(ref, k):
        r0 = pl.multiple_of(k * _STRIP, _STRIP)
        return ref[pl.ds(r0, _STRIP), :]

    _accum_strips(p_ref, l_ref, acc_ref, index, n_strips, _LANES)

    @pl.when(step == pl.num_programs(1) - 1)
    def _finish():
        num_ref[0, 0] = jnp.sum(acc_ref[0]) * w_l1
        den_ref[0, 0] = jnp.sum(acc_ref[1])


def _native_path(p, l, w_l1):
    n, _, h, w = p.shape
    imgs = next(b for b in (8, 4, 2, 1) if n % b == 0)
    tiles = n // imgs

    def in_map(i):
        return (i, 0, 0, 0)

    body = functools.partial(
        _native_kernel, strips_per_img=h // _STRIP, h=h, w=w, imgs=imgs,
        w_l1=w_l1)
    total = n * h * w
    cost = pl.CostEstimate(flops=6 * total, transcendentals=0,
                           bytes_accessed=8 * total)

    loss = pl.pallas_call(
        body,
        grid=(tiles,),
        in_specs=[
            pl.BlockSpec((imgs, None, h, w), in_map),
            pl.BlockSpec((imgs, None, h, w), in_map),
        ],
        out_specs=pl.BlockSpec(memory_space=pltpu.SMEM),
        out_shape=jax.ShapeDtypeStruct((1, 1), jnp.float32),
        scratch_shapes=[pltpu.VMEM((2, _SUBLANES, _LANES), jnp.float32)],
        compiler_params=pltpu.CompilerParams(
            dimension_semantics=("arbitrary",)),
        cost_estimate=cost,
    )(p, l)
    return {'losses': {'depth_loss': loss[0, 0]}}


def _flat_path(p, l, w_l1):
    total = p.size
    p = p.reshape(-1)
    l = l.reshape(-1)

    span = _CORES * _ROW_QUANT * _LANES
    padded = pl.cdiv(total, span) * span
    if padded != total:
        p = jnp.pad(p, (0, padded - total))
        l = jnp.pad(l, (0, padded - total))

    rows = padded // _LANES
    rows_per_core = rows // _CORES
    row_tile = next(t for t in _TILE_CANDIDATES if rows_per_core % t == 0)
    tiles_per_core = rows_per_core // row_tile
    p2 = p.reshape(rows, _LANES)
    l2 = l.reshape(rows, _LANES)

    def in_map(c, i):
        return (c * tiles_per_core + i, 0)

    body = functools.partial(_flat_kernel, n_strips=row_tile // _STRIP,
                             w_l1=w_l1)
    out_specs, out_shape = _scalar_outputs()
    cost = pl.CostEstimate(flops=6 * total, transcendentals=0,
                           bytes_accessed=8 * total)

    num_p, den_p = pl.pallas_call(
        body,
        grid=(_CORES, tiles_per_core),
        in_specs=[
            pl.BlockSpec((row_tile, _LANES), in_map),
            pl.BlockSpec((row_tile, _LANES), in_map),
        ],
        out_specs=out_specs,
        out_shape=out_shape,
        scratch_shapes=[pltpu.VMEM((2, _SUBLANES, _LANES), jnp.float32)],
        compiler_params=pltpu.CompilerParams(
            dimension_semantics=("parallel", "arbitrary")),
        cost_estimate=cost,
    )(p2, l2)
    return _finalize(num_p, den_p, w_l1)


def kernel(depth_preds, depth_labels):
    w_l1 = 1.0

    shape = depth_preds.shape
    if (len(shape) == 4
            and shape[1] == 1
            and shape[-1] % _LANES == 0
            and shape[-2] % _STRIP == 0
            and shape[0] % _CORES == 0):
        return _native_path(depth_preds, depth_labels, w_l1)
    return _flat_path(depth_preds, depth_labels, w_l1)


# speedup vs baseline: 1.1374x; 1.1374x over previous
import functools
import math

import jax
import jax.numpy as jnp
from jax.experimental import pallas as pl
from jax.experimental.pallas import tpu as pltpu

_LANES = 128
_SUBLANES = 8
_STRIP = 64
_CORES = 2
_ROW_QUANT = 2048
_TILE_CANDIDATES = (9216, 8192, 6144, 4608, 4096, 3072, 2048)


def _scalar_outputs():
    return (
        [
            pl.BlockSpec((None, 1, 1), lambda c, i: (c, 0, 0),
                         memory_space=pltpu.SMEM),
            pl.BlockSpec((None, 1, 1), lambda c, i: (c, 0, 0),
                         memory_space=pltpu.SMEM),
        ],
        [
            jax.ShapeDtypeStruct((_CORES, 1, 1), jnp.float32),
            jax.ShapeDtypeStruct((_CORES, 1, 1), jnp.float32),
        ],
    )


def _finalize(num_p, den_p, w_l1):
    num = num_p[0, 0, 0] + num_p[1, 0, 0]
    den = den_p[0, 0, 0] + den_p[1, 0, 0]
    return {'losses': {'depth_loss': (w_l1 * num) / (den + 1e-6)}}


def _accum_strips(p_ref, l_ref, acc_ref, index, n_strips, width):
    folds = _STRIP // _SUBLANES
    lgroups = width // _LANES

    def fold(x):
        s = x.reshape(folds, _SUBLANES, width).sum(axis=0)
        if lgroups == 1:
            return s
        return sum(s[:, g * _LANES:(g + 1) * _LANES] for g in range(lgroups))

    def strip(k, carry):
        acc_n, acc_d = carry
        p = index(p_ref, k).astype(jnp.float32)
        l = index(l_ref, k).astype(jnp.float32)
        valid = l > 1e-6
        l1 = jnp.where(valid, jnp.abs(p - l), 0.0)
        one = jnp.where(valid, 1.0, 0.0)
        return acc_n + fold(l1), acc_d + fold(one)

    zero = jnp.zeros((_SUBLANES, _LANES), jnp.float32)
    acc_n, acc_d = jax.lax.fori_loop(0, n_strips, strip, (zero, zero))
    acc_ref[0] += acc_n
    acc_ref[1] += acc_d


def _native_kernel(p_ref, l_ref, loss_ref, acc_ref, *,
                   strips_per_img, h, w, imgs, w_l1):
    step = pl.program_id(0)

    @pl.when(step == 0)
    def _init():
        acc_ref[...] = jnp.zeros_like(acc_ref)

    folds = _STRIP // _SUBLANES
    lgroups = w // _LANES

    def fold(x):
        s = x.reshape(folds, _SUBLANES, w).sum(axis=0)
        if lgroups == 1:
            return s
        return sum(s[:, g * _LANES:(g + 1) * _LANES] for g in range(lgroups))

    def img_body(i, carry):
        acc_n, acc_d = carry
        for s in range(strips_per_img):
            r0 = s * _STRIP
            p = p_ref[i, r0:r0 + _STRIP, :].astype(jnp.float32)
            l = l_ref[i, r0:r0 + _STRIP, :].astype(jnp.float32)
            valid = l > 1e-6
            l1 = jnp.where(valid, jnp.abs(p - l), 0.0)
            one = jnp.where(valid, 1.0, 0.0)
            acc_n = acc_n + fold(l1)
            acc_d = acc_d + fold(one)
        return acc_n, acc_d

    zero = jnp.zeros((_SUBLANES, _LANES), jnp.float32)
    acc_n, acc_d = jax.lax.fori_loop(0, imgs, img_body, (zero, zero))
    acc_ref[0] += acc_n
    acc_ref[1] += acc_d

    @pl.when(step == pl.num_programs(0) - 1)
    def _finish():
        num = jnp.sum(acc_ref[0]) * w_l1
        den = jnp.sum(acc_ref[1])
        loss_ref[0, 0] = num / (den + 1e-6)


def _flat_kernel(p_ref, l_ref, num_ref, den_ref, acc_ref, *,
                 n_strips, w_l1):
    step = pl.program_id(1)

    @pl.when(step == 0)
    def _init():
        acc_ref[...] = jnp.zeros_like(acc_ref)

    def index(ref, k):
        r0 = pl.multiple_of(k * _STRIP, _STRIP)
        return ref[pl.ds(r0, _STRIP), :]

    _accum_strips(p_ref, l_ref, acc_ref, index, n_strips, _LANES)

    @pl.when(step == pl.num_programs(1) - 1)
    def _finish():
        num_ref[0, 0] = jnp.sum(acc_ref[0]) * w_l1
        den_ref[0, 0] = jnp.sum(acc_ref[1])


def _native_path(p, l, w_l1):
    n, _, h, w = p.shape
    imgs = next(b for b in (8, 4, 2, 1) if n % b == 0)
    tiles = n // imgs

    def in_map(i):
        return (i, 0, 0, 0)

    body = functools.partial(
        _native_kernel, strips_per_img=h // _STRIP, h=h, w=w, imgs=imgs,
        w_l1=w_l1)
    total = n * h * w
    cost = pl.CostEstimate(flops=6 * total, transcendentals=0,
                           bytes_accessed=8 * total)

    loss = pl.pallas_call(
        body,
        grid=(tiles,),
        in_specs=[
            pl.BlockSpec((imgs, None, h, w), in_map),
            pl.BlockSpec((imgs, None, h, w), in_map),
        ],
        out_specs=pl.BlockSpec(memory_space=pltpu.SMEM),
        out_shape=jax.ShapeDtypeStruct((1, 1), jnp.float32),
        scratch_shapes=[pltpu.VMEM((2, _SUBLANES, _LANES), jnp.float32)],
        compiler_params=pltpu.CompilerParams(
            dimension_semantics=("arbitrary",)),
        cost_estimate=cost,
    )(p, l)
    return {'losses': {'depth_loss': loss[0, 0]}}


def _flat_path(p, l, w_l1):
    total = p.size
    p = p.reshape(-1)
    l = l.reshape(-1)

    span = _CORES * _ROW_QUANT * _LANES
    padded = pl.cdiv(total, span) * span
    if padded != total:
        p = jnp.pad(p, (0, padded - total))
        l = jnp.pad(l, (0, padded - total))

    rows = padded // _LANES
    rows_per_core = rows // _CORES
    row_tile = next(t for t in _TILE_CANDIDATES if rows_per_core % t == 0)
    tiles_per_core = rows_per_core // row_tile
    p2 = p.reshape(rows, _LANES)
    l2 = l.reshape(rows, _LANES)

    def in_map(c, i):
        return (c * tiles_per_core + i, 0)

    body = functools.partial(_flat_kernel, n_strips=row_tile // _STRIP,
                             w_l1=w_l1)
    out_specs, out_shape = _scalar_outputs()
    cost = pl.CostEstimate(flops=6 * total, transcendentals=0,
                           bytes_accessed=8 * total)

    num_p, den_p = pl.pallas_call(
        body,
        grid=(_CORES, tiles_per_core),
        in_specs=[
            pl.BlockSpec((row_tile, _LANES), in_map),
            pl.BlockSpec((row_tile, _LANES), in_map),
        ],
        out_specs=out_specs,
        out_shape=out_shape,
        scratch_shapes=[pltpu.VMEM((2, _SUBLANES, _LANES), jnp.float32)],
        compiler_params=pltpu.CompilerParams(
            dimension_semantics=("parallel", "arbitrary")),
        cost_estimate=cost,
    )(p2, l2)
    return _finalize(num_p, den_p, w_l1)


def kernel(depth_preds, depth_labels):
    w_l1 = 1.0

    shape = depth_preds.shape
    if (len(shape) == 4
            and shape[1] == 1
            and shape[-1] % _LANES == 0
            and shape[-2] % _STRIP == 0
            and shape[0] % _CORES == 0):
        return _native_path(depth_preds, depth_labels, w_l1)
    return _flat_path(depth_preds, depth_labels, w_l1)


# speedup vs baseline: 1.1500x; 1.0111x over previous
import functools
import math

import jax
import jax.numpy as jnp
from jax.experimental import pallas as pl
from jax.experimental.pallas import tpu as pltpu

_LANES = 128
_SUBLANES = 8
_STRIP = 64
_CORES = 2
_ROW_QUANT = 2048
_TILE_CANDIDATES = (9216, 8192, 6144, 4608, 4096, 3072, 2048)


def _scalar_outputs():
    return (
        [
            pl.BlockSpec((None, 1, 1), lambda c, i: (c, 0, 0),
                         memory_space=pltpu.SMEM),
            pl.BlockSpec((None, 1, 1), lambda c, i: (c, 0, 0),
                         memory_space=pltpu.SMEM),
        ],
        [
            jax.ShapeDtypeStruct((_CORES, 1, 1), jnp.float32),
            jax.ShapeDtypeStruct((_CORES, 1, 1), jnp.float32),
        ],
    )


def _finalize(num_p, den_p, w_l1):
    num = num_p[0, 0, 0] + num_p[1, 0, 0]
    den = den_p[0, 0, 0] + den_p[1, 0, 0]
    return {'losses': {'depth_loss': (w_l1 * num) / (den + 1e-6)}}


def _accum_strips(p_ref, l_ref, acc_ref, index, n_strips, width):
    folds = _STRIP // _SUBLANES
    lgroups = width // _LANES

    def fold(x):
        s = x.reshape(folds, _SUBLANES, width).sum(axis=0)
        if lgroups == 1:
            return s
        return sum(s[:, g * _LANES:(g + 1) * _LANES] for g in range(lgroups))

    def strip(k, carry):
        acc_n, acc_d = carry
        p = index(p_ref, k).astype(jnp.float32)
        l = index(l_ref, k).astype(jnp.float32)
        valid = l > 1e-6
        l1 = jnp.where(valid, jnp.abs(p - l), 0.0)
        one = jnp.where(valid, 1.0, 0.0)
        return acc_n + fold(l1), acc_d + fold(one)

    zero = jnp.zeros((_SUBLANES, _LANES), jnp.float32)
    acc_n, acc_d = jax.lax.fori_loop(0, n_strips, strip, (zero, zero))
    acc_ref[0] += acc_n
    acc_ref[1] += acc_d


def _native_kernel(p_ref, l_ref, loss_ref, acc_ref, *,
                   strips_per_img, w, imgs, w_l1):
    step = pl.program_id(0)

    @pl.when(step == 0)
    def _init():
        acc_ref[...] = jnp.zeros_like(acc_ref)

    folds = _STRIP // _SUBLANES
    lgroups = w // _LANES

    def fold(x):
        s = x.reshape(folds, _SUBLANES, w).sum(axis=0)
        if lgroups == 1:
            return s
        return sum(s[:, g * _LANES:(g + 1) * _LANES] for g in range(lgroups))

    def img_body(i, carry):
        acc_n, acc_d = carry
        for s in range(strips_per_img):
            r0 = s * _STRIP
            p = p_ref[i, r0:r0 + _STRIP, :].astype(jnp.float32)
            l = l_ref[i, r0:r0 + _STRIP, :].astype(jnp.float32)
            valid = l > 1e-6
            l1 = jnp.where(valid, jnp.abs(p - l), 0.0)
            one = jnp.where(valid, 1.0, 0.0)
            acc_n = acc_n + fold(l1)
            acc_d = acc_d + fold(one)
        return acc_n, acc_d

    zero = jnp.zeros((_SUBLANES, _LANES), jnp.float32)
    acc_n, acc_d = jax.lax.fori_loop(0, imgs, img_body, (zero, zero))
    acc_ref[0] += acc_n
    acc_ref[1] += acc_d

    @pl.when(step == pl.num_programs(0) - 1)
    def _finish():
        num = jnp.sum(acc_ref[0]) * w_l1
        den = jnp.sum(acc_ref[1])
        loss_ref[0, 0] = num / (den + 1e-6)


def _flat_kernel(p_ref, l_ref, num_ref, den_ref, acc_ref, *,
                 n_strips, w_l1):
    step = pl.program_id(1)

    @pl.when(step == 0)
    def _init():
        acc_ref[...] = jnp.zeros_like(acc_ref)

    def index(ref, k):
        r0 = pl.multiple_of(k * _STRIP, _STRIP)
        return ref[pl.ds(r0, _STRIP), :]

    _accum_strips(p_ref, l_ref, acc_ref, index, n_strips, _LANES)

    @pl.when(step == pl.num_programs(1) - 1)
    def _finish():
        num_ref[0, 0] = jnp.sum(acc_ref[0]) * w_l1
        den_ref[0, 0] = jnp.sum(acc_ref[1])


def _native_path(p, l, w_l1):
    n, _, h, w = p.shape
    imgs = next(b for b in (8, 4, 2, 1) if n % b == 0)
    tiles = n // imgs

    def in_map(i):
        return (i, 0, 0, 0)

    body = functools.partial(
        _native_kernel, strips_per_img=h // _STRIP, w=w, imgs=imgs,
        w_l1=w_l1)
    total = n * h * w
    cost = pl.CostEstimate(flops=6 * total, transcendentals=0,
                           bytes_accessed=8 * total)

    loss = pl.pallas_call(
        body,
        grid=(tiles,),
        in_specs=[
            pl.BlockSpec((imgs, None, h, w), in_map),
            pl.BlockSpec((imgs, None, h, w), in_map),
        ],
        out_specs=pl.BlockSpec(memory_space=pltpu.SMEM),
        out_shape=jax.ShapeDtypeStruct((1, 1), jnp.float32),
        scratch_shapes=[pltpu.VMEM((2, _SUBLANES, _LANES), jnp.float32)],
        compiler_params=pltpu.CompilerParams(
            dimension_semantics=("arbitrary",)),
        cost_estimate=cost,
    )(p, l)
    return {'losses': {'depth_loss': loss[0, 0]}}


def _flat_path(p, l, w_l1):
    total = p.size
    p = p.reshape(-1)
    l = l.reshape(-1)

    span = _CORES * _ROW_QUANT * _LANES
    padded = pl.cdiv(total, span) * span
    if padded != total:
        p = jnp.pad(p, (0, padded - total))
        l = jnp.pad(l, (0, padded - total))

    rows = padded // _LANES
    rows_per_core = rows // _CORES
    row_tile = next(t for t in _TILE_CANDIDATES if rows_per_core % t == 0)
    tiles_per_core = rows_per_core // row_tile
    p2 = p.reshape(rows, _LANES)
    l2 = l.reshape(rows, _LANES)

    def in_map(c, i):
        return (c * tiles_per_core + i, 0)

    body = functools.partial(_flat_kernel, n_strips=row_tile // _STRIP,
                             w_l1=w_l1)
    out_specs, out_shape = _scalar_outputs()
    cost = pl.CostEstimate(flops=6 * total, transcendentals=0,
                           bytes_accessed=8 * total)

    num_p, den_p = pl.pallas_call(
        body,
        grid=(_CORES, tiles_per_core),
        in_specs=[
            pl.BlockSpec((row_tile, _LANES), in_map),
            pl.BlockSpec((row_tile, _LANES), in_map),
        ],
        out_specs=out_specs,
        out_shape=out_shape,
        scratch_shapes=[pltpu.VMEM((2, _SUBLANES, _LANES), jnp.float32)],
        compiler_params=pltpu.CompilerParams(
            dimension_semantics=("parallel", "arbitrary")),
        cost_estimate=cost,
    )(p2, l2)
    return _finalize(num_p, den_p, w_l1)


def kernel(depth_preds, depth_labels):
    w_l1 = 1.0

    shape = depth_preds.shape
    if (len(shape) == 4
            and shape[1] == 1
            and shape[-1] % _LANES == 0
            and shape[-2] % _STRIP == 0
            and shape[0] % _CORES == 0):
        return _native_path(depth_preds, depth_labels, w_l1)
    return _flat_path(depth_preds, depth_labels, w_l1)
